# baseline (speedup 1.0000x reference)
import functools

import jax
import jax.numpy as jnp
from jax import lax
from jax.experimental import pallas as pl
from jax.experimental.pallas import tpu as pltpu
from jax.experimental.pallas import tpu_sc as plsc

N = 10000
D = 128
NP = 10112
RPS = NP // 16
NC, NS = 2, 16
CPT = 80
K = 128
GS = 8
NG = CPT // GS
EPAD = NC * NS * CPT * K
RB = 2528


def _mesh():
    return plsc.VectorSubcoreMesh(
        core_axis_name="c", subcore_axis_name="s", num_cores=NC,
        num_subcores=NS)


def _deg_kernel(dst_idx, ones, zeros):
    @functools.partial(
        pl.kernel,
        out_type=jax.ShapeDtypeStruct((NC, NP, D), jnp.float32),
        mesh=_mesh(),
        scratch_types=[
            pltpu.VMEM((CPT, K), jnp.int32),
            pltpu.VMEM((K, D), jnp.float32),
            pltpu.VMEM_SHARED((NP, D), jnp.float32),
        ],
    )
    def body(dst_h, ones_h, zeros_h, out_h, dstbuf, onesbuf, sdeg):
        c = lax.axis_index("c")
        s = lax.axis_index("s")
        rows = pl.ds(s * RPS, RPS)

        @pl.when(c == 0)
        def _():
            pltpu.sync_copy(ones_h.at[rows], sdeg.at[rows])

        @pl.when(c != 0)
        def _():
            pltpu.sync_copy(zeros_h.at[rows], sdeg.at[rows])

        pltpu.sync_copy(dst_h.at[c].at[s], dstbuf)
        pltpu.sync_copy(ones_h.at[pl.ds(0, K)], onesbuf)
        plsc.subcore_barrier()

        @pl.loop(0, CPT)
        def _(j):
            pltpu.sync_copy(onesbuf, sdeg.at[dstbuf.at[j]], add=True)

        plsc.subcore_barrier()
        pltpu.sync_copy(sdeg.at[rows], out_h.at[c].at[rows])

    return body(dst_idx, ones, zeros)


def _agg_kernel(src_idx, dst_idx, h, zeros):
    @functools.partial(
        pl.kernel,
        out_type=jax.ShapeDtypeStruct((NC, NP, D), jnp.float32),
        mesh=_mesh(),
        scratch_types=[
            pltpu.VMEM((GS, K), jnp.int32),
            pltpu.VMEM((GS, K), jnp.int32),
            pltpu.VMEM((K, D), jnp.float32),
            pltpu.VMEM((K, D), jnp.float32),
            pltpu.SemaphoreType.DMA,
            pltpu.SemaphoreType.DMA,
            pltpu.SemaphoreType.DMA,
            pltpu.SemaphoreType.DMA,
            pltpu.VMEM_SHARED((NP, D), jnp.float32),
        ],
    )
    def body(src_h, dst_h, h_h, z_h, out_h,
             sibuf, dibuf, buf0, buf1, sem0, sem1, sem2, sem3, sagg):
        c = lax.axis_index("c")
        s = lax.axis_index("s")
        rows = pl.ds(s * RPS, RPS)

        @pl.when(c == 0)
        def _():
            pltpu.sync_copy(h_h.at[rows], sagg.at[rows])

        @pl.when(c != 0)
        def _():
            pltpu.sync_copy(z_h.at[rows], sagg.at[rows])

        plsc.subcore_barrier()

        def gather(j, buf, sem):
            pltpu.async_copy(h_h.at[sibuf.at[j]], buf, sem)

        def gwait(j, buf, sem):
            pltpu.make_async_copy(h_h.at[sibuf.at[j]], buf, sem).wait()

        def scat(j, buf, sem):
            pltpu.async_copy(buf, sagg.at[dibuf.at[j]], sem, add=True)

        def swait(j, buf, sem):
            pltpu.make_async_copy(buf, sagg.at[dibuf.at[j]], sem).wait()

        @pl.loop(0, NG)
        def _(g):
            blk = pl.ds(g * GS, GS)
            pltpu.sync_copy(src_h.at[c].at[s].at[blk], sibuf)
            pltpu.sync_copy(dst_h.at[c].at[s].at[blk], dibuf)
            gather(0, buf0, sem0)
            gather(1, buf1, sem1)
            for k in range(GS // 2 - 1):
                j = 2 * k
                gwait(j, buf0, sem0)
                scat(j, buf0, sem2)
                gwait(j + 1, buf1, sem1)
                scat(j + 1, buf1, sem3)
                swait(j, buf0, sem2)
                gather(j + 2, buf0, sem0)
                swait(j + 1, buf1, sem3)
                gather(j + 3, buf1, sem1)
            gwait(GS - 2, buf0, sem0)
            scat(GS - 2, buf0, sem2)
            gwait(GS - 1, buf1, sem1)
            scat(GS - 1, buf1, sem3)
            swait(GS - 2, buf0, sem2)
            swait(GS - 1, buf1, sem3)

        plsc.subcore_barrier()
        pltpu.sync_copy(sagg.at[rows], out_h.at[c].at[rows])

    return body(src_idx, dst_idx, h, zeros)


def _mm_body(x_ref, w_ref, degp_ref, o_ref, dis_ref):
    deg = degp_ref[0, :, 0:1] + degp_ref[1, :, 0:1]
    dis = lax.rsqrt(deg)
    h = lax.dot_general(x_ref[...], w_ref[...], (((1,), (0,)), ((), ())),
                        precision=lax.Precision.HIGHEST,
                        preferred_element_type=jnp.float32)
    o_ref[...] = h * dis
    dis_ref[...] = jnp.broadcast_to(dis, dis_ref.shape)


def _cmb_mm_body(p_ref, dis_ref, b_ref, w_ref, o_ref):
    dis = dis_ref[:, 0:1]
    y = jnp.maximum((p_ref[0] + p_ref[1]) * dis + b_ref[...], 0.0)
    h = lax.dot_general(y, w_ref[...], (((1,), (0,)), ((), ())),
                        precision=lax.Precision.HIGHEST,
                        preferred_element_type=jnp.float32)
    o_ref[...] = h * dis


def _out_body(p_ref, dis_ref, b_ref, o_ref):
    o_ref[...] = jnp.maximum(
        (p_ref[0] + p_ref[1]) * dis_ref[:, 0:1] + b_ref[...], 0.0)


TB = 2000
_row_spec = pl.BlockSpec((TB, D), lambda i: (i, 0))
_w_spec = pl.BlockSpec((D, D), lambda i: (0, 0))
_degp_spec = pl.BlockSpec((2, TB, D), lambda i: (0, i, 0))
_p_spec = pl.BlockSpec((2, TB, D), lambda i: (0, i, 0))
_dis_spec = pl.BlockSpec((TB, 8), lambda i: (i, 0))
_b_spec = pl.BlockSpec((1, D), lambda i: (0, 0))
_grid = (N // TB,)


def _mm(x, w, degp):
    return pl.pallas_call(
        _mm_body, grid=_grid,
        in_specs=[_row_spec, _w_spec, _degp_spec],
        out_specs=(_row_spec, _dis_spec),
        out_shape=(jax.ShapeDtypeStruct((NP, D), jnp.float32),
                   jax.ShapeDtypeStruct((N, 8), jnp.float32)))(x, w, degp)


def _cmb_mm(p, dis, b, w):
    return pl.pallas_call(
        _cmb_mm_body, grid=_grid,
        in_specs=[_p_spec, _dis_spec, _b_spec, _w_spec],
        out_specs=_row_spec,
        out_shape=jax.ShapeDtypeStruct((NP, D), jnp.float32))(p, dis, b, w)


def _out(p, dis, b):
    return pl.pallas_call(
        _out_body, grid=_grid,
        in_specs=[_p_spec, _dis_spec, _b_spec],
        out_specs=_row_spec,
        out_shape=jax.ShapeDtypeStruct((N, D), jnp.float32))(p, dis, b)


def kernel(x, edge_index, W1, b1, W2, b2):
    ei = edge_index.astype(jnp.int32)
    pad = EPAD - ei.shape[1]
    pad_src = (jnp.arange(pad, dtype=jnp.int32) * 37) % N
    pad_dst = N + (jnp.arange(pad, dtype=jnp.int32) % (NP - N))
    src = jnp.concatenate([ei[0], pad_src]).reshape(NC, NS, CPT, K)
    dst = jnp.concatenate([ei[1], pad_dst]).reshape(NC, NS, CPT, K)

    zeros = jnp.zeros((NP, D), jnp.float32)
    ones = jnp.ones((NP, D), jnp.float32)
    b1r = b1.reshape(1, D)
    b2r = b2.reshape(1, D)

    degp = _deg_kernel(dst, ones, zeros)

    h1, dis = _mm(x, W1, degp)
    p1 = _agg_kernel(src, dst, h1, zeros)
    h2 = _cmb_mm(p1, dis, b1r, W2)
    p2 = _agg_kernel(src, dst, h2, zeros)
    return _out(p2, dis, b2r)


# speedup vs baseline: 1.1572x; 1.1572x over previous
import functools

import jax
import jax.numpy as jnp
from jax import lax
from jax.experimental import pallas as pl
from jax.experimental.pallas import tpu as pltpu
from jax.experimental.pallas import tpu_sc as plsc

N = 10000
D = 128
NP = 10112
RPS = NP // 16
NC, NS = 2, 16
CPT = 80
K = 128
GS = 8
NG = CPT // GS
EPAD = NC * NS * CPT * K
RB = 2528


def _mesh():
    return plsc.VectorSubcoreMesh(
        core_axis_name="c", subcore_axis_name="s", num_cores=NC,
        num_subcores=NS)


def _fill(buf, rows_, value):
    vec = jnp.full((16,), value, jnp.float32)
    for r in range(rows_):
        for l in range(D // 16):
            buf[r, pl.ds(l * 16, 16)] = vec


def _seed(zbuf, acc, s):
    for r in range(RPS // 64):
        pltpu.sync_copy(zbuf.at[pl.ds(0, 64)],
                        acc.at[pl.ds(s * RPS + r * 64, 64)])
    rem = RPS % 64
    if rem:
        pltpu.sync_copy(zbuf.at[pl.ds(0, rem)],
                        acc.at[pl.ds(s * RPS + (RPS // 64) * 64, rem)])


def _deg_kernel(dst_idx):
    @functools.partial(
        pl.kernel,
        out_type=jax.ShapeDtypeStruct((NC, NP, D), jnp.float32),
        mesh=_mesh(),
        scratch_types=[
            pltpu.VMEM((CPT, K), jnp.int32),
            pltpu.VMEM((K, D), jnp.float32),
            pltpu.VMEM((64, D), jnp.float32),
            pltpu.VMEM_SHARED((NP, D), jnp.float32),
        ],
    )
    def body(dst_h, out_h, dstbuf, onesbuf, zbuf, sdeg):
        c = lax.axis_index("c")
        s = lax.axis_index("s")
        rows = pl.ds(s * RPS, RPS)

        pltpu.sync_copy(dst_h.at[c].at[s], dstbuf)
        _fill(onesbuf, K, 1.0)
        _fill(zbuf, 64, 0.0)
        _seed(zbuf, sdeg, s)
        plsc.subcore_barrier()

        @pl.loop(0, CPT)
        def _(j):
            pltpu.sync_copy(onesbuf, sdeg.at[dstbuf.at[j]], add=True)

        plsc.subcore_barrier()
        pltpu.sync_copy(sdeg.at[rows], out_h.at[c].at[rows])

    return body(dst_idx)


def _agg_kernel(src_idx, dst_idx, h):
    @functools.partial(
        pl.kernel,
        out_type=jax.ShapeDtypeStruct((NC, NP, D), jnp.float32),
        mesh=_mesh(),
        scratch_types=[
            pltpu.VMEM((GS, K), jnp.int32),
            pltpu.VMEM((GS, K), jnp.int32),
            pltpu.VMEM((K, D), jnp.float32),
            pltpu.VMEM((K, D), jnp.float32),
            pltpu.SemaphoreType.DMA,
            pltpu.SemaphoreType.DMA,
            pltpu.VMEM_SHARED((NP, D), jnp.float32),
        ],
    )
    def body(src_h, dst_h, h_h, out_h,
             sibuf, dibuf, buf0, buf1, sem0, sem1, sagg):
        c = lax.axis_index("c")
        s = lax.axis_index("s")
        rows = pl.ds(s * RPS, RPS)

        @pl.when(c == 0)
        def _():
            pltpu.sync_copy(h_h.at[rows], sagg.at[rows])

        @pl.when(c != 0)
        def _():
            _fill(buf0, 64, 0.0)
            _seed(buf0, sagg, s)

        plsc.subcore_barrier()

        def gather(j, buf, sem):
            pltpu.async_copy(h_h.at[sibuf.at[j]], buf, sem)

        def gwait(j, buf, sem):
            pltpu.make_async_copy(h_h.at[sibuf.at[j]], buf, sem).wait()

        def scat(j, buf):
            pltpu.sync_copy(buf, sagg.at[dibuf.at[j]], add=True)

        @pl.loop(0, NG)
        def _(g):
            blk = pl.ds(g * GS, GS)
            pltpu.sync_copy(src_h.at[c].at[s].at[blk], sibuf)
            pltpu.sync_copy(dst_h.at[c].at[s].at[blk], dibuf)
            gather(0, buf0, sem0)
            for k in range(GS // 2 - 1):
                j = 2 * k
                gather(j + 1, buf1, sem1)
                gwait(j, buf0, sem0)
                scat(j, buf0)
                gather(j + 2, buf0, sem0)
                gwait(j + 1, buf1, sem1)
                scat(j + 1, buf1)
            gather(GS - 1, buf1, sem1)
            gwait(GS - 2, buf0, sem0)
            scat(GS - 2, buf0)
            gwait(GS - 1, buf1, sem1)
            scat(GS - 1, buf1)

        plsc.subcore_barrier()
        pltpu.sync_copy(sagg.at[rows], out_h.at[c].at[rows])

    return body(src_idx, dst_idx, h)


def _mm_body(x_ref, w_ref, degp_ref, o_ref, dis_ref):
    deg = degp_ref[0, :, 0:1] + degp_ref[1, :, 0:1]
    dis = lax.rsqrt(deg + 1.0)
    h = lax.dot_general(x_ref[...], w_ref[...], (((1,), (0,)), ((), ())),
                        precision=lax.Precision.HIGHEST,
                        preferred_element_type=jnp.float32)
    o_ref[...] = h * dis
    dis_ref[...] = jnp.broadcast_to(dis, dis_ref.shape)


def _cmb_mm_body(p_ref, dis_ref, b_ref, w_ref, o_ref):
    dis = dis_ref[:, 0:1]
    y = jnp.maximum((p_ref[0] + p_ref[1]) * dis + b_ref[...], 0.0)
    h = lax.dot_general(y, w_ref[...], (((1,), (0,)), ((), ())),
                        precision=lax.Precision.HIGHEST,
                        preferred_element_type=jnp.float32)
    o_ref[...] = h * dis


def _out_body(p_ref, dis_ref, b_ref, o_ref):
    o_ref[...] = jnp.maximum(
        (p_ref[0] + p_ref[1]) * dis_ref[:, 0:1] + b_ref[...], 0.0)


TB = 2000
_row_spec = pl.BlockSpec((TB, D), lambda i: (i, 0))
_w_spec = pl.BlockSpec((D, D), lambda i: (0, 0))
_degp_spec = pl.BlockSpec((2, TB, D), lambda i: (0, i, 0))
_p_spec = pl.BlockSpec((2, TB, D), lambda i: (0, i, 0))
_dis_spec = pl.BlockSpec((TB, 8), lambda i: (i, 0))
_b_spec = pl.BlockSpec((1, D), lambda i: (0, 0))
_grid = (N // TB,)


def _mm(x, w, degp):
    return pl.pallas_call(
        _mm_body, grid=_grid,
        in_specs=[_row_spec, _w_spec, _degp_spec],
        out_specs=(_row_spec, _dis_spec),
        out_shape=(jax.ShapeDtypeStruct((NP, D), jnp.float32),
                   jax.ShapeDtypeStruct((N, 8), jnp.float32)))(x, w, degp)


def _cmb_mm(p, dis, b, w):
    return pl.pallas_call(
        _cmb_mm_body, grid=_grid,
        in_specs=[_p_spec, _dis_spec, _b_spec, _w_spec],
        out_specs=_row_spec,
        out_shape=jax.ShapeDtypeStruct((NP, D), jnp.float32))(p, dis, b, w)


def _out(p, dis, b):
    return pl.pallas_call(
        _out_body, grid=_grid,
        in_specs=[_p_spec, _dis_spec, _b_spec],
        out_specs=_row_spec,
        out_shape=jax.ShapeDtypeStruct((N, D), jnp.float32))(p, dis, b)


def kernel(x, edge_index, W1, b1, W2, b2):
    ei = edge_index.astype(jnp.int32)
    pad = EPAD - ei.shape[1]
    pad_src = (jnp.arange(pad, dtype=jnp.int32) * 37) % N
    pad_dst = N + (jnp.arange(pad, dtype=jnp.int32) % (NP - N))
    src = jnp.concatenate([ei[0], pad_src]).reshape(NC, NS, CPT, K)
    dst = jnp.concatenate([ei[1], pad_dst]).reshape(NC, NS, CPT, K)

    b1r = b1.reshape(1, D)
    b2r = b2.reshape(1, D)

    degp = _deg_kernel(dst)

    h1, dis = _mm(x, W1, degp)
    p1 = _agg_kernel(src, dst, h1)
    h2 = _cmb_mm(p1, dis, b1r, W2)
    p2 = _agg_kernel(src, dst, h2)
    return _out(p2, dis, b2r)


# speedup vs baseline: 1.2901x; 1.1149x over previous
import functools

import jax
import jax.numpy as jnp
from jax import lax
from jax.experimental import pallas as pl
from jax.experimental.pallas import tpu as pltpu
from jax.experimental.pallas import tpu_sc as plsc

N = 10000
D = 128
NP = 10112
RPS = NP // 16
NC, NS = 2, 16
CPT = 80
K = 128
GS = 40
NG = CPT // GS
EPAD = NC * NS * CPT * K
RB = 2528


def _mesh():
    return plsc.VectorSubcoreMesh(
        core_axis_name="c", subcore_axis_name="s", num_cores=NC,
        num_subcores=NS)


def _fill(buf, rows_, value):
    vec = jnp.full((16,), value, jnp.float32)
    for r in range(rows_):
        for l in range(D // 16):
            buf[r, pl.ds(l * 16, 16)] = vec


def _seed(zbuf, acc, s):
    for r in range(RPS // 64):
        pltpu.sync_copy(zbuf.at[pl.ds(0, 64)],
                        acc.at[pl.ds(s * RPS + r * 64, 64)])
    rem = RPS % 64
    if rem:
        pltpu.sync_copy(zbuf.at[pl.ds(0, rem)],
                        acc.at[pl.ds(s * RPS + (RPS // 64) * 64, rem)])


def _deg_kernel(dst_idx):
    @functools.partial(
        pl.kernel,
        out_type=jax.ShapeDtypeStruct((NC, NP, D), jnp.float32),
        mesh=_mesh(),
        scratch_types=[
            pltpu.VMEM((CPT, K), jnp.int32),
            pltpu.VMEM((K, D), jnp.float32),
            pltpu.VMEM((64, D), jnp.float32),
            pltpu.VMEM_SHARED((NP, D), jnp.float32),
        ],
    )
    def body(dst_h, out_h, dstbuf, onesbuf, zbuf, sdeg):
        c = lax.axis_index("c")
        s = lax.axis_index("s")
        rows = pl.ds(s * RPS, RPS)

        pltpu.sync_copy(dst_h.at[c].at[s], dstbuf)
        _fill(onesbuf, K, 1.0)
        _fill(zbuf, 64, 0.0)
        _seed(zbuf, sdeg, s)
        plsc.subcore_barrier()

        @pl.loop(0, CPT)
        def _(j):
            pltpu.sync_copy(onesbuf, sdeg.at[dstbuf.at[j]], add=True)

        plsc.subcore_barrier()
        pltpu.sync_copy(sdeg.at[rows], out_h.at[c].at[rows])

    return body(dst_idx)


def _agg_kernel(src_idx, dst_idx, h):
    @functools.partial(
        pl.kernel,
        out_type=jax.ShapeDtypeStruct((NC, NP, D), jnp.float32),
        mesh=_mesh(),
        scratch_types=[
            pltpu.VMEM((GS, K), jnp.int32),
            pltpu.VMEM((GS, K), jnp.int32),
            pltpu.VMEM((K, D), jnp.float32),
            pltpu.VMEM((K, D), jnp.float32),
            pltpu.SemaphoreType.DMA,
            pltpu.SemaphoreType.DMA,
            pltpu.VMEM_SHARED((NP, D), jnp.float32),
        ],
    )
    def body(src_h, dst_h, h_h, out_h,
             sibuf, dibuf, buf0, buf1, sem0, sem1, sagg):
        c = lax.axis_index("c")
        s = lax.axis_index("s")
        rows = pl.ds(s * RPS, RPS)

        @pl.when(c == 0)
        def _():
            pltpu.sync_copy(h_h.at[rows], sagg.at[rows])

        @pl.when(c != 0)
        def _():
            _fill(buf0, 64, 0.0)
            _seed(buf0, sagg, s)

        plsc.subcore_barrier()

        def gather(j, buf, sem):
            pltpu.async_copy(h_h.at[sibuf.at[j]], buf, sem)

        def gwait(j, buf, sem):
            pltpu.make_async_copy(h_h.at[sibuf.at[j]], buf, sem).wait()

        def scat(j, buf):
            pltpu.sync_copy(buf, sagg.at[dibuf.at[j]], add=True)

        for g in range(NG):
            blk = pl.ds(g * GS, GS)
            pltpu.sync_copy(src_h.at[c].at[s].at[blk], sibuf)
            pltpu.sync_copy(dst_h.at[c].at[s].at[blk], dibuf)
            gather(0, buf0, sem0)

            @pl.loop(0, GS // 2 - 1)
            def _(k):
                j = 2 * k
                gather(j + 1, buf1, sem1)
                gwait(j, buf0, sem0)
                scat(j, buf0)
                gather(j + 2, buf0, sem0)
                gwait(j + 1, buf1, sem1)
                scat(j + 1, buf1)

            gather(GS - 1, buf1, sem1)
            gwait(GS - 2, buf0, sem0)
            scat(GS - 2, buf0)
            gwait(GS - 1, buf1, sem1)
            scat(GS - 1, buf1)

        plsc.subcore_barrier()
        pltpu.sync_copy(sagg.at[rows], out_h.at[c].at[rows])

    return body(src_idx, dst_idx, h)


def _mm_body(x_ref, w_ref, degp_ref, o_ref, dis_ref):
    deg = degp_ref[0, :, 0:1] + degp_ref[1, :, 0:1]
    dis = lax.rsqrt(deg + 1.0)
    h = lax.dot_general(x_ref[...], w_ref[...], (((1,), (0,)), ((), ())),
                        precision=lax.Precision.HIGHEST,
                        preferred_element_type=jnp.float32)
    o_ref[...] = h * dis
    dis_ref[...] = jnp.broadcast_to(dis, dis_ref.shape)


def _cmb_mm_body(p_ref, dis_ref, b_ref, w_ref, o_ref):
    dis = dis_ref[:, 0:1]
    y = jnp.maximum((p_ref[0] + p_ref[1]) * dis + b_ref[...], 0.0)
    h = lax.dot_general(y, w_ref[...], (((1,), (0,)), ((), ())),
                        precision=lax.Precision.HIGHEST,
                        preferred_element_type=jnp.float32)
    o_ref[...] = h * dis


def _out_body(p_ref, dis_ref, b_ref, o_ref):
    o_ref[...] = jnp.maximum(
        (p_ref[0] + p_ref[1]) * dis_ref[:, 0:1] + b_ref[...], 0.0)


TB = 2000
_row_spec = pl.BlockSpec((TB, D), lambda i: (i, 0))
_w_spec = pl.BlockSpec((D, D), lambda i: (0, 0))
_degp_spec = pl.BlockSpec((2, TB, D), lambda i: (0, i, 0))
_p_spec = pl.BlockSpec((2, TB, D), lambda i: (0, i, 0))
_dis_spec = pl.BlockSpec((TB, 8), lambda i: (i, 0))
_b_spec = pl.BlockSpec((1, D), lambda i: (0, 0))
_grid = (N // TB,)


def _mm(x, w, degp):
    return pl.pallas_call(
        _mm_body, grid=_grid,
        in_specs=[_row_spec, _w_spec, _degp_spec],
        out_specs=(_row_spec, _dis_spec),
        out_shape=(jax.ShapeDtypeStruct((NP, D), jnp.float32),
                   jax.ShapeDtypeStruct((N, 8), jnp.float32)))(x, w, degp)


def _cmb_mm(p, dis, b, w):
    return pl.pallas_call(
        _cmb_mm_body, grid=_grid,
        in_specs=[_p_spec, _dis_spec, _b_spec, _w_spec],
        out_specs=_row_spec,
        out_shape=jax.ShapeDtypeStruct((NP, D), jnp.float32))(p, dis, b, w)


def _out(p, dis, b):
    return pl.pallas_call(
        _out_body, grid=_grid,
        in_specs=[_p_spec, _dis_spec, _b_spec],
        out_specs=_row_spec,
        out_shape=jax.ShapeDtypeStruct((N, D), jnp.float32))(p, dis, b)


def kernel(x, edge_index, W1, b1, W2, b2):
    ei = edge_index.astype(jnp.int32)
    pad = EPAD - ei.shape[1]
    pad_src = (jnp.arange(pad, dtype=jnp.int32) * 37) % N
    pad_dst = N + (jnp.arange(pad, dtype=jnp.int32) % (NP - N))
    src = jnp.concatenate([ei[0], pad_src]).reshape(NC, NS, CPT, K)
    dst = jnp.concatenate([ei[1], pad_dst]).reshape(NC, NS, CPT, K)

    b1r = b1.reshape(1, D)
    b2r = b2.reshape(1, D)

    degp = _deg_kernel(dst)

    h1, dis = _mm(x, W1, degp)
    p1 = _agg_kernel(src, dst, h1)
    h2 = _cmb_mm(p1, dis, b1r, W2)
    p2 = _agg_kernel(src, dst, h2)
    return _out(p2, dis, b2r)


# speedup vs baseline: 1.2906x; 1.0004x over previous
import functools

import jax
import jax.numpy as jnp
from jax import lax
from jax.experimental import pallas as pl
from jax.experimental.pallas import tpu as pltpu
from jax.experimental.pallas import tpu_sc as plsc

N = 10000
D = 128
NP = 10112
RPS = NP // 16
NC, NS = 2, 16
CPT = 80
K = 128
GS = 40
NG = CPT // GS
EPAD = NC * NS * CPT * K
RB = 2528


def _mesh():
    return plsc.VectorSubcoreMesh(
        core_axis_name="c", subcore_axis_name="s", num_cores=NC,
        num_subcores=NS)


def _fill(buf, rows_, value):
    vec = jnp.full((16,), value, jnp.float32)
    for r in range(rows_):
        for l in range(D // 16):
            buf[r, pl.ds(l * 16, 16)] = vec


def _seed(zbuf, acc, s):
    for r in range(RPS // 64):
        pltpu.sync_copy(zbuf.at[pl.ds(0, 64)],
                        acc.at[pl.ds(s * RPS + r * 64, 64)])
    rem = RPS % 64
    if rem:
        pltpu.sync_copy(zbuf.at[pl.ds(0, rem)],
                        acc.at[pl.ds(s * RPS + (RPS // 64) * 64, rem)])


def _deg_kernel(dst_idx):
    @functools.partial(
        pl.kernel,
        out_type=jax.ShapeDtypeStruct((NC, NP, D), jnp.float32),
        mesh=_mesh(),
        scratch_types=[
            pltpu.VMEM((CPT, K), jnp.int32),
            pltpu.VMEM((K, D), jnp.float32),
            pltpu.VMEM((64, D), jnp.float32),
            pltpu.SemaphoreType.DMA,
            pltpu.VMEM_SHARED((NP, D), jnp.float32),
        ],
    )
    def body(dst_h, out_h, dstbuf, onesbuf, zbuf, sem, sdeg):
        c = lax.axis_index("c")
        s = lax.axis_index("s")
        rows = pl.ds(s * RPS, RPS)
        W = 8

        pltpu.sync_copy(dst_h.at[c].at[s], dstbuf)
        _fill(onesbuf, K, 1.0)
        _fill(zbuf, 64, 0.0)
        _seed(zbuf, sdeg, s)
        plsc.subcore_barrier()

        @pl.loop(0, CPT)
        def _(j):
            pltpu.async_copy(onesbuf, sdeg.at[dstbuf.at[j]], sem, add=True)

            @pl.when(j >= W)
            def _():
                pltpu.make_async_copy(
                    onesbuf, sdeg.at[dstbuf.at[j - W]], sem).wait()

        @pl.loop(CPT - W, CPT)
        def _(j):
            pltpu.make_async_copy(onesbuf, sdeg.at[dstbuf.at[j]], sem).wait()

        plsc.subcore_barrier()
        pltpu.sync_copy(sdeg.at[rows], out_h.at[c].at[rows])

    return body(dst_idx)


def _agg_kernel(src_idx, dst_idx, h):
    @functools.partial(
        pl.kernel,
        out_type=jax.ShapeDtypeStruct((NC, NP, D), jnp.float32),
        mesh=_mesh(),
        scratch_types=[
            pltpu.VMEM((GS, K), jnp.int32),
            pltpu.VMEM((GS, K), jnp.int32),
            pltpu.VMEM((K, D), jnp.float32),
            pltpu.VMEM((K, D), jnp.float32),
            pltpu.SemaphoreType.DMA,
            pltpu.SemaphoreType.DMA,
            pltpu.VMEM_SHARED((NP, D), jnp.float32),
        ],
    )
    def body(src_h, dst_h, h_h, out_h,
             sibuf, dibuf, buf0, buf1, sem0, sem1, sagg):
        c = lax.axis_index("c")
        s = lax.axis_index("s")
        rows = pl.ds(s * RPS, RPS)

        @pl.when(c == 0)
        def _():
            pltpu.sync_copy(h_h.at[rows], sagg.at[rows])

        @pl.when(c != 0)
        def _():
            _fill(buf0, 64, 0.0)
            _seed(buf0, sagg, s)

        plsc.subcore_barrier()

        def gather(j, buf, sem):
            pltpu.async_copy(h_h.at[sibuf.at[j]], buf, sem)

        def gwait(j, buf, sem):
            pltpu.make_async_copy(h_h.at[sibuf.at[j]], buf, sem).wait()

        def scat(j, buf):
            pltpu.sync_copy(buf, sagg.at[dibuf.at[j]], add=True)

        for g in range(NG):
            blk = pl.ds(g * GS, GS)
            pltpu.sync_copy(src_h.at[c].at[s].at[blk], sibuf)
            pltpu.sync_copy(dst_h.at[c].at[s].at[blk], dibuf)
            gather(0, buf0, sem0)

            @pl.loop(0, GS // 2 - 1)
            def _(k):
                j = 2 * k
                gather(j + 1, buf1, sem1)
                gwait(j, buf0, sem0)
                scat(j, buf0)
                gather(j + 2, buf0, sem0)
                gwait(j + 1, buf1, sem1)
                scat(j + 1, buf1)

            gather(GS - 1, buf1, sem1)
            gwait(GS - 2, buf0, sem0)
            scat(GS - 2, buf0)
            gwait(GS - 1, buf1, sem1)
            scat(GS - 1, buf1)

        plsc.subcore_barrier()
        pltpu.sync_copy(sagg.at[rows], out_h.at[c].at[rows])

    return body(src_idx, dst_idx, h)


def _mm_body(x_ref, w_ref, degp_ref, o_ref, dis_ref):
    deg = degp_ref[0, :, 0:1] + degp_ref[1, :, 0:1]
    dis = lax.rsqrt(deg + 1.0)
    h = lax.dot_general(x_ref[...], w_ref[...], (((1,), (0,)), ((), ())),
                        precision=lax.Precision.HIGHEST,
                        preferred_element_type=jnp.float32)
    o_ref[...] = h * dis
    dis_ref[...] = jnp.broadcast_to(dis, dis_ref.shape)


def _cmb_mm_body(p_ref, dis_ref, b_ref, w_ref, o_ref):
    dis = dis_ref[:, 0:1]
    y = jnp.maximum((p_ref[0] + p_ref[1]) * dis + b_ref[...], 0.0)
    h = lax.dot_general(y, w_ref[...], (((1,), (0,)), ((), ())),
                        precision=lax.Precision.HIGHEST,
                        preferred_element_type=jnp.float32)
    o_ref[...] = h * dis


def _out_body(p_ref, dis_ref, b_ref, o_ref):
    o_ref[...] = jnp.maximum(
        (p_ref[0] + p_ref[1]) * dis_ref[:, 0:1] + b_ref[...], 0.0)


TB = 2000
_row_spec = pl.BlockSpec((TB, D), lambda i: (i, 0))
_w_spec = pl.BlockSpec((D, D), lambda i: (0, 0))
_degp_spec = pl.BlockSpec((2, TB, D), lambda i: (0, i, 0))
_p_spec = pl.BlockSpec((2, TB, D), lambda i: (0, i, 0))
_dis_spec = pl.BlockSpec((TB, 8), lambda i: (i, 0))
_b_spec = pl.BlockSpec((1, D), lambda i: (0, 0))
_grid = (N // TB,)


def _mm(x, w, degp):
    return pl.pallas_call(
        _mm_body, grid=_grid,
        in_specs=[_row_spec, _w_spec, _degp_spec],
        out_specs=(_row_spec, _dis_spec),
        out_shape=(jax.ShapeDtypeStruct((NP, D), jnp.float32),
                   jax.ShapeDtypeStruct((N, 8), jnp.float32)))(x, w, degp)


def _cmb_mm(p, dis, b, w):
    return pl.pallas_call(
        _cmb_mm_body, grid=_grid,
        in_specs=[_p_spec, _dis_spec, _b_spec, _w_spec],
        out_specs=_row_spec,
        out_shape=jax.ShapeDtypeStruct((NP, D), jnp.float32))(p, dis, b, w)


def _out(p, dis, b):
    return pl.pallas_call(
        _out_body, grid=_grid,
        in_specs=[_p_spec, _dis_spec, _b_spec],
        out_specs=_row_spec,
        out_shape=jax.ShapeDtypeStruct((N, D), jnp.float32))(p, dis, b)


def kernel(x, edge_index, W1, b1, W2, b2):
    ei = edge_index.astype(jnp.int32)
    pad = EPAD - ei.shape[1]
    pad_src = (jnp.arange(pad, dtype=jnp.int32) * 37) % N
    pad_dst = N + (jnp.arange(pad, dtype=jnp.int32) % (NP - N))
    src = jnp.concatenate([ei[0], pad_src]).reshape(NC, NS, CPT, K)
    dst = jnp.concatenate([ei[1], pad_dst]).reshape(NC, NS, CPT, K)

    b1r = b1.reshape(1, D)
    b2r = b2.reshape(1, D)

    degp = _deg_kernel(dst)

    h1, dis = _mm(x, W1, degp)
    p1 = _agg_kernel(src, dst, h1)
    h2 = _cmb_mm(p1, dis, b1r, W2)
    p2 = _agg_kernel(src, dst, h2)
    return _out(p2, dis, b2r)
